# baseline (speedup 1.0000x reference)
import functools

import jax
import jax.numpy as jnp
from jax import lax
from jax.experimental import pallas as pl
from jax.experimental.pallas import tpu as pltpu
from jax.experimental.pallas import tpu_sc as plsc

_B = 16384
_EMB = 32
_CHUNK = 128


def _make_gather(nc, ns):
  nw = nc * ns
  b_per_w = _B // nw
  n_chunks = b_per_w // _CHUNK
  mesh = plsc.VectorSubcoreMesh(core_axis_name="c", subcore_axis_name="s")

  @functools.partial(
      pl.kernel,
      mesh=mesh,
      compiler_params=pltpu.CompilerParams(use_tc_tiling_on_sc=False),
      out_type=(
          jax.ShapeDtypeStruct((nw, n_chunks, _CHUNK, _EMB), jnp.float32),
          jax.ShapeDtypeStruct((nw, n_chunks, _CHUNK, _EMB), jnp.float32),
      ),
      scratch_types=[
          pltpu.VMEM((n_chunks, _CHUNK), jnp.int32),
          pltpu.VMEM((n_chunks, _CHUNK), jnp.int32),
          pltpu.VMEM((n_chunks, _CHUNK, _EMB), jnp.float32),
          pltpu.VMEM((n_chunks, _CHUNK, _EMB), jnp.float32),
          pltpu.SemaphoreType.DMA,
          pltpu.SemaphoreType.DMA,
      ],
  )
  def gather_k(uid_hbm, mid_hbm, uemb_hbm, memb_hbm, uout_hbm, mout_hbm,
               uidx_v, midx_v, urows_v, mrows_v, usem, msem):
    wid = lax.axis_index("s") * nc + lax.axis_index("c")
    pltpu.sync_copy(uid_hbm.at[wid], uidx_v)
    pltpu.sync_copy(mid_hbm.at[wid], midx_v)
    ucopies = []
    mcopies = []
    for j in range(n_chunks):
      ucopies.append(pltpu.async_copy(uemb_hbm.at[uidx_v.at[j]],
                                      urows_v.at[j], usem))
      mcopies.append(pltpu.async_copy(memb_hbm.at[midx_v.at[j]],
                                      mrows_v.at[j], msem))
    for c in ucopies:
      c.wait()
    pltpu.sync_copy(urows_v, uout_hbm.at[wid])
    for c in mcopies:
      c.wait()
    pltpu.sync_copy(mrows_v, mout_hbm.at[wid])

  return gather_k, nw, n_chunks


def _mlp_body(uf, mf, w1, b1, w2, b2, w3, b3, w4, b4, out):
  hp = lax.Precision.HIGHEST
  x = jnp.dot(uf[...], w1[0:_EMB, :], precision=hp)
  x = x + jnp.dot(mf[...], w1[_EMB:2 * _EMB, :], precision=hp)
  x = jnp.maximum(x + b1[...], 0.0)
  x = jnp.maximum(jnp.dot(x, w2[...], precision=hp) + b2[...], 0.0)
  x = jnp.maximum(jnp.dot(x, w3[...], precision=hp) + b3[...], 0.0)
  x = jnp.dot(x, w4[...], precision=hp) + b4[...]
  out[...] = 5.0 / (1.0 + jnp.exp(-x)) + 1.0


def kernel(user_id, movie_id, user_emb, movie_emb, W1, b1, W2, b2, W3, b3,
           W4, b4):
  info = plsc.get_sparse_core_info()
  gather_k, nw, n_chunks = _make_gather(info.num_cores, info.num_subcores)

  uid = user_id.astype(jnp.int32).reshape(nw, n_chunks, _CHUNK)
  mid = movie_id.astype(jnp.int32).reshape(nw, n_chunks, _CHUNK)
  uf4, mf4 = gather_k(uid, mid, user_emb, movie_emb)
  uf = uf4.reshape(_B, _EMB)
  mf = mf4.reshape(_B, _EMB)

  blk = 2048
  grid = (_B // blk,)
  feat_spec = pl.BlockSpec((blk, _EMB), lambda i: (i, 0))
  out = pl.pallas_call(
      _mlp_body,
      grid=grid,
      in_specs=[
          feat_spec,
          feat_spec,
          pl.BlockSpec((2 * _EMB, 32), lambda i: (0, 0)),
          pl.BlockSpec((1, 32), lambda i: (0, 0)),
          pl.BlockSpec((32, 16), lambda i: (0, 0)),
          pl.BlockSpec((1, 16), lambda i: (0, 0)),
          pl.BlockSpec((16, 8), lambda i: (0, 0)),
          pl.BlockSpec((1, 8), lambda i: (0, 0)),
          pl.BlockSpec((8, 1), lambda i: (0, 0)),
          pl.BlockSpec((1, 1), lambda i: (0, 0)),
      ],
      out_specs=pl.BlockSpec((blk, 1), lambda i: (i, 0)),
      out_shape=jax.ShapeDtypeStruct((_B, 1), jnp.float32),
  )(uf, mf, W1, b1.reshape(1, 32), W2, b2.reshape(1, 16), W3,
    b3.reshape(1, 8), W4, b4.reshape(1, 1))
  return out


# speedup vs baseline: 1.0448x; 1.0448x over previous
import functools

import jax
import jax.numpy as jnp
from jax import lax
from jax.experimental import pallas as pl
from jax.experimental.pallas import tpu as pltpu
from jax.experimental.pallas import tpu_sc as plsc

_B = 16384
_EMB = 32
_CHUNK = 128
_VIEW = 4


def _make_gather(nc, ns, n_rows):
  nw = nc * ns
  b_per_w = _B // nw
  n_chunks = b_per_w // _CHUNK
  mesh = plsc.VectorSubcoreMesh(core_axis_name="c", subcore_axis_name="s")

  @functools.partial(
      pl.kernel,
      mesh=mesh,
      compiler_params=pltpu.CompilerParams(needs_layout_passes=False),
      out_type=(
          jax.ShapeDtypeStruct((_EMB, _B), jnp.float32),
          jax.ShapeDtypeStruct((_EMB, _B), jnp.float32),
      ),
      scratch_types=[
          pltpu.VMEM((b_per_w,), jnp.int32),
          pltpu.VMEM((b_per_w,), jnp.int32),
          pltpu.VMEM((n_chunks, _CHUNK), jnp.int32),
          pltpu.VMEM((n_chunks, _CHUNK), jnp.int32),
          pltpu.VMEM((2, _CHUNK, 4 * _EMB), jnp.float32),
          pltpu.VMEM((2, _CHUNK, 4 * _EMB), jnp.float32),
          pltpu.VMEM((_EMB, b_per_w), jnp.float32),
          pltpu.VMEM((_EMB, b_per_w), jnp.float32),
          pltpu.SemaphoreType.DMA,
          pltpu.SemaphoreType.DMA,
          pltpu.SemaphoreType.DMA,
          pltpu.SemaphoreType.DMA,
      ],
  )
  def gather_k(uid_hbm, mid_hbm, uembv_hbm, membv_hbm, ufT_hbm, mfT_hbm,
               uidx_v, midx_v, uvidx_v, mvidx_v, uraw_v, mraw_v,
               ufT_v, mfT_v, us0, us1, ms0, ms1):
    usems = (us0, us1)
    msems = (ms0, ms1)
    wid = lax.axis_index("s") * nc + lax.axis_index("c")
    base = wid * b_per_w
    pltpu.sync_copy(uid_hbm.at[wid], uidx_v)
    pltpu.sync_copy(mid_hbm.at[wid], midx_v)

    for j in range(n_chunks):
      def build(g, _, j=j):
        s_src = pl.ds(j * _CHUNK + g * 16, 16)
        s_dst = pl.ds(g * 16, 16)
        uvidx_v[j, s_dst] = jnp.right_shift(uidx_v[s_src], 2)
        mvidx_v[j, s_dst] = jnp.right_shift(midx_v[s_src], 2)
        return 0
      lax.fori_loop(0, _CHUNK // 16, build, 0)

    def fire(j):
      slot = j % 2
      cu = pltpu.async_copy(uembv_hbm.at[uvidx_v.at[j]], uraw_v.at[slot],
                            usems[slot])
      cm = pltpu.async_copy(membv_hbm.at[mvidx_v.at[j]], mraw_v.at[slot],
                            msems[slot])
      return cu, cm

    copies = {0: fire(0)}
    for j in range(n_chunks):
      if j + 1 < n_chunks:
        copies[j + 1] = fire(j + 1)
      cu, cm = copies.pop(j)
      cu.wait()
      cm.wait()
      slot = j % 2

      def select(g, _, j=j, slot=slot):
        r16 = lax.iota(jnp.int32, 16) + g * 16
        rg = r16 + j * _CHUNK
        uidx16 = uidx_v[pl.ds(j * _CHUNK + g * 16, 16)]
        midx16 = midx_v[pl.ds(j * _CHUNK + g * 16, 16)]
        uoff = jnp.left_shift(jnp.bitwise_and(uidx16, _VIEW - 1), 5)
        moff = jnp.left_shift(jnp.bitwise_and(midx16, _VIEW - 1), 5)
        for c in range(_EMB):
          csplat = jnp.full((16,), c, jnp.int32)
          uvals = plsc.load_gather(uraw_v.at[slot], [r16, uoff + c])
          plsc.store_scatter(ufT_v, [csplat, rg], uvals)
          mvals = plsc.load_gather(mraw_v.at[slot], [r16, moff + c])
          plsc.store_scatter(mfT_v, [csplat, rg], mvals)
        return 0

      lax.fori_loop(0, _CHUNK // 16, select, 0)

    pltpu.sync_copy(ufT_v, ufT_hbm.at[:, pl.ds(base, b_per_w)])
    pltpu.sync_copy(mfT_v, mfT_hbm.at[:, pl.ds(base, b_per_w)])

  return gather_k, nw


def _mlp_body(ufT, mfT, w1, b1, w2, b2, w3, b3, w4, b4, out):
  hp = lax.Precision.HIGHEST
  dn = (((0,), (0,)), ((), ()))
  h = lax.dot_general(w1[0:_EMB, :], ufT[...], dn, precision=hp)
  h = h + lax.dot_general(w1[_EMB:2 * _EMB, :], mfT[...], dn, precision=hp)
  h = jnp.maximum(h + b1[...], 0.0)
  h = jnp.maximum(lax.dot_general(w2[...], h, dn, precision=hp) + b2[...],
                  0.0)
  h = jnp.maximum(lax.dot_general(w3[...], h, dn, precision=hp) + b3[...],
                  0.0)
  h = lax.dot_general(w4[...], h, dn, precision=hp) + b4[...]
  out[...] = 5.0 / (1.0 + jnp.exp(-h)) + 1.0


def kernel(user_id, movie_id, user_emb, movie_emb, W1, b1, W2, b2, W3, b3,
           W4, b4):
  info = plsc.get_sparse_core_info()
  n_rows = user_emb.shape[0]
  gather_k, nw = _make_gather(info.num_cores, info.num_subcores, n_rows)

  uid = user_id.astype(jnp.int32).reshape(nw, _B // nw)
  mid = movie_id.astype(jnp.int32).reshape(nw, _B // nw)
  uemb_v = user_emb.reshape(n_rows // _VIEW, _VIEW * _EMB)
  memb_v = movie_emb.reshape(n_rows // _VIEW, _VIEW * _EMB)
  ufT, mfT = gather_k(uid, mid, uemb_v, memb_v)

  out = pl.pallas_call(
      _mlp_body,
      grid=(1,),
      in_specs=[
          pl.BlockSpec((_EMB, _B), lambda i: (0, 0)),
          pl.BlockSpec((_EMB, _B), lambda i: (0, 0)),
          pl.BlockSpec((2 * _EMB, 32), lambda i: (0, 0)),
          pl.BlockSpec((32, 1), lambda i: (0, 0)),
          pl.BlockSpec((32, 16), lambda i: (0, 0)),
          pl.BlockSpec((16, 1), lambda i: (0, 0)),
          pl.BlockSpec((16, 8), lambda i: (0, 0)),
          pl.BlockSpec((8, 1), lambda i: (0, 0)),
          pl.BlockSpec((8, 1), lambda i: (0, 0)),
          pl.BlockSpec((1, 1), lambda i: (0, 0)),
      ],
      out_specs=pl.BlockSpec((1, _B), lambda i: (0, 0)),
      out_shape=jax.ShapeDtypeStruct((1, _B), jnp.float32),
  )(ufT, mfT, W1, b1.reshape(32, 1), W2, b2.reshape(16, 1), W3,
    b3.reshape(8, 1), W4, b4.reshape(1, 1))
  return out.reshape(_B, 1)


# speedup vs baseline: 2.9692x; 2.8419x over previous
import functools

import jax
import jax.numpy as jnp
from jax import lax
from jax.experimental import pallas as pl
from jax.experimental.pallas import tpu as pltpu
from jax.experimental.pallas import tpu_sc as plsc

_B = 16384
_EMB = 32
_ROWS = 1000000
_KPAD = 8192
_FLAT = _KPAD * _EMB * 128
_CK = 128
_JBLK = (_ROWS + _CK * 128 - 1) // (_CK * 128)


def _linearize_body(u_ref, m_ref, ou_ref, om_ref):
  for v in range(_CK):
    sl = pl.ds(v * 128, 128)
    ou_ref[v] = u_ref[:, sl]
    om_ref[v] = m_ref[:, sl]


def _linearize(uT, mT):
  return pl.pallas_call(
      _linearize_body,
      grid=(_EMB // 8, _JBLK),
      in_specs=[
          pl.BlockSpec((8, _CK * 128), lambda t, j: (t, j)),
          pl.BlockSpec((8, _CK * 128), lambda t, j: (t, j)),
      ],
      out_specs=[
          pl.BlockSpec((_CK, 8, 128), lambda t, j: (j, t, 0)),
          pl.BlockSpec((_CK, 8, 128), lambda t, j: (j, t, 0)),
      ],
      out_shape=[
          jax.ShapeDtypeStruct((_KPAD, _EMB, 128), jnp.float32),
          jax.ShapeDtypeStruct((_KPAD, _EMB, 128), jnp.float32),
      ],
  )(uT, mT)


def _make_gather(nc, ns):
  nw = nc * ns
  b_per_w = _B // nw
  mesh = plsc.VectorSubcoreMesh(core_axis_name="c", subcore_axis_name="s")
  span = ((_ROWS + 127) // 128 - 1) * 4096 + 128

  @functools.partial(
      pl.kernel,
      mesh=mesh,
      compiler_params=pltpu.CompilerParams(use_tc_tiling_on_sc=False),
      out_type=(
          jax.ShapeDtypeStruct((_EMB, _B), jnp.float32),
          jax.ShapeDtypeStruct((_EMB, _B), jnp.float32),
      ),
      scratch_types=[
          pltpu.VMEM((b_per_w,), jnp.int32),
          pltpu.VMEM((b_per_w,), jnp.int32),
          pltpu.VMEM((b_per_w,), jnp.int32),
          pltpu.VMEM((b_per_w,), jnp.int32),
          pltpu.VMEM((_EMB, b_per_w), jnp.float32),
          pltpu.VMEM((_EMB, b_per_w), jnp.float32),
          pltpu.SemaphoreType.DMA,
          pltpu.SemaphoreType.DMA,
      ],
  )
  def gather_k(uid_hbm, mid_hbm, uflat_hbm, mflat_hbm, ufT_hbm, mfT_hbm,
               uidx_v, midx_v, ubase_v, mbase_v, udata_v, mdata_v,
               usem, msem):
    wid = lax.axis_index("s") * nc + lax.axis_index("c")
    base = wid * b_per_w
    pltpu.sync_copy(uid_hbm.at[wid], uidx_v)
    pltpu.sync_copy(mid_hbm.at[wid], midx_v)

    def mkbase(g, _):
      s = pl.ds(g * 16, 16)
      u = uidx_v[s]
      ubase_v[s] = jnp.left_shift(jnp.right_shift(u, 7), 12) | (u & 127)
      m = midx_v[s]
      mbase_v[s] = jnp.left_shift(jnp.right_shift(m, 7), 12) | (m & 127)
      return 0

    lax.fori_loop(0, b_per_w // 16, mkbase, 0)

    copies = []
    lag = 8
    for c in range(_EMB):
      cu = pltpu.async_copy(
          uflat_hbm.at[pl.ds(c * 128, span)].at[ubase_v], udata_v.at[c],
          usem)
      cm = pltpu.async_copy(
          mflat_hbm.at[pl.ds(c * 128, span)].at[mbase_v], mdata_v.at[c],
          msem)
      copies.append((cu, cm))
      if c >= lag:
        copies[c - lag][0].wait()
        copies[c - lag][1].wait()
    for c in range(_EMB - lag, _EMB):
      copies[c][0].wait()
      copies[c][1].wait()

    pltpu.sync_copy(udata_v, ufT_hbm.at[:, pl.ds(base, b_per_w)])
    pltpu.sync_copy(mdata_v, mfT_hbm.at[:, pl.ds(base, b_per_w)])

  return gather_k, nw


def _mlp_body(ufT, mfT, w1, b1, w2, b2, w3, b3, w4, b4, out):
  hp = lax.Precision.HIGHEST
  dn = (((0,), (0,)), ((), ()))
  h = lax.dot_general(w1[0:_EMB, :], ufT[...], dn, precision=hp)
  h = h + lax.dot_general(w1[_EMB:2 * _EMB, :], mfT[...], dn, precision=hp)
  h = jnp.maximum(h + b1[...], 0.0)
  h = jnp.maximum(lax.dot_general(w2[...], h, dn, precision=hp) + b2[...],
                  0.0)
  h = jnp.maximum(lax.dot_general(w3[...], h, dn, precision=hp) + b3[...],
                  0.0)
  h = lax.dot_general(w4[...], h, dn, precision=hp) + b4[...]
  out[...] = 5.0 / (1.0 + jnp.exp(-h)) + 1.0


def kernel(user_id, movie_id, user_emb, movie_emb, W1, b1, W2, b2, W3, b3,
           W4, b4):
  info = plsc.get_sparse_core_info()
  gather_k, nw = _make_gather(info.num_cores, info.num_subcores)

  uL, mL = _linearize(user_emb.T, movie_emb.T)
  uflat = uL.reshape(_FLAT)
  mflat = mL.reshape(_FLAT)

  uid = user_id.astype(jnp.int32).reshape(nw, _B // nw)
  mid = movie_id.astype(jnp.int32).reshape(nw, _B // nw)
  ufT, mfT = gather_k(uid, mid, uflat, mflat)

  out = pl.pallas_call(
      _mlp_body,
      grid=(1,),
      in_specs=[
          pl.BlockSpec((_EMB, _B), lambda i: (0, 0)),
          pl.BlockSpec((_EMB, _B), lambda i: (0, 0)),
          pl.BlockSpec((2 * _EMB, 32), lambda i: (0, 0)),
          pl.BlockSpec((32, 1), lambda i: (0, 0)),
          pl.BlockSpec((32, 16), lambda i: (0, 0)),
          pl.BlockSpec((16, 1), lambda i: (0, 0)),
          pl.BlockSpec((16, 8), lambda i: (0, 0)),
          pl.BlockSpec((8, 1), lambda i: (0, 0)),
          pl.BlockSpec((8, 1), lambda i: (0, 0)),
          pl.BlockSpec((1, 1), lambda i: (0, 0)),
      ],
      out_specs=pl.BlockSpec((1, _B), lambda i: (0, 0)),
      out_shape=jax.ShapeDtypeStruct((1, _B), jnp.float32),
  )(ufT, mfT, W1, b1.reshape(32, 1), W2, b2.reshape(16, 1), W3,
    b3.reshape(8, 1), W4, b4.reshape(1, 1))
  return out.reshape(_B, 1)
